# baseline (speedup 1.0000x reference)
import jax
from jax.experimental import pallas as pl

_ROWS = 1024


def _copy_block(t_ref, o_ref):
    o_ref[...] = t_ref[...]


def kernel(x, table):
    del x
    T, E = table.shape
    out = pl.pallas_call(
        _copy_block,
        grid=(T // _ROWS,),
        in_specs=[pl.BlockSpec((_ROWS, E), lambda i: (i, 0))],
        out_specs=pl.BlockSpec((_ROWS, E), lambda i: (i, 0)),
        out_shape=jax.ShapeDtypeStruct((T, E), table.dtype),
    )(table)
    return out[None, :, :]
